# baseline (speedup 1.0000x reference)
import functools

import jax
import jax.numpy as jnp
from jax import lax
from jax.experimental import pallas as pl
from jax.experimental.pallas import tpu as pltpu
from jax.experimental.pallas import tpu_sc as plsc

T = 1024
D = 512
N = 50000
P = 64
K_NEAREST = 4
TN = 3584
NPAD = 50176
GRID = NPAD // TN
HR = NPAD // 128


def _tc_body(h_ref, f_ref, ppr_ref, gr_ref, pph_ref, gh_ref, ph_ref, tgt_ref,
             idx_out, val_out, hn_ref, q_ref, bv_ref, bi_ref):
    i = pl.program_id(0)
    tgt = tgt_ref[0, 0]

    @pl.when(i == 0)
    def _prologue():
        h = h_ref[...]
        hn_ref[...] = h / (jnp.sqrt(jnp.sum(h * h, axis=1, keepdims=True)) + 1e-8)
        bv_ref[...] = jnp.full((T, 1), -jnp.inf, jnp.float32)
        bi_ref[...] = jnp.zeros((T, 1), jnp.int32)
        gvm = gh_ref[...] == tgt
        ppm = jnp.where(gvm, pph_ref[...], -1)
        li1 = lax.broadcasted_iota(jnp.int32, (1, 128), 1)
        hrow = jnp.zeros((1, 128), jnp.float32)
        for p in range(P):
            cnt = jnp.sum((ppm == p).astype(jnp.float32))
            hrow = hrow + jnp.where(li1 == p, cnt, 0.0)
        ph = ph_ref[...]
        li = lax.broadcasted_iota(jnp.int32, (T, 128), 1)
        oh = (li == ph).astype(jnp.float32)
        counts = lax.dot_general(oh, hrow, (((1,), (1,)), ((), ())),
                                 preferred_element_type=jnp.float32)
        up = counts >= K_NEAREST
        q_ref[...] = jnp.where(up, oh, (li == P).astype(jnp.float32)).astype(jnp.bfloat16)

    f = f_ref[...]
    cn = f / (jnp.sqrt(jnp.sum(f * f, axis=1, keepdims=True)) + 1e-8)
    sim = lax.dot_general(hn_ref[...], cn, (((1,), (1,)), ((), ())),
                          preferred_element_type=jnp.float32)
    ppr = ppr_ref[0:1, pl.ds(i * TN, TN)]
    gvr = gr_ref[0:1, pl.ds(i * TN, TN)] == tgt
    ch = lax.broadcasted_iota(jnp.int32, (128, TN), 0)
    code = (((ppr == ch) | (ch == P)) & gvr).astype(jnp.bfloat16)
    msk = lax.dot_general(q_ref[...], code, (((1,), (0,)), ((), ())),
                          preferred_element_type=jnp.float32)
    simm = jnp.where(msk > 0.5, sim, -jnp.inf)
    tmax = jnp.max(simm, axis=1, keepdims=True)
    tidx = jnp.argmax(simm, axis=1).astype(jnp.int32).reshape(T, 1) + i * TN
    better = tmax > bv_ref[...]
    bv_ref[...] = jnp.where(better, tmax, bv_ref[...])
    bi_ref[...] = jnp.where(better, tidx, bi_ref[...])

    @pl.when(i == GRID - 1)
    def _epilogue():
        valid = bv_ref[...] > -jnp.inf
        idx_out[...] = jnp.where(valid, bi_ref[...], 0)
        val_out[...] = jnp.broadcast_to(valid.astype(jnp.float32), (T, 16))


def _tc_argmax(h_clean, features, pp_row, g_row, pp_hist, g_hist, phones2, tgt):
    return pl.pallas_call(
        _tc_body,
        grid=(GRID,),
        in_specs=[
            pl.BlockSpec((T, D), lambda i: (0, 0)),
            pl.BlockSpec((TN, D), lambda i: (i, 0)),
            pl.BlockSpec((1, NPAD), lambda i: (0, 0)),
            pl.BlockSpec((1, NPAD), lambda i: (0, 0)),
            pl.BlockSpec((HR, 128), lambda i: (0, 0)),
            pl.BlockSpec((HR, 128), lambda i: (0, 0)),
            pl.BlockSpec((T, 1), lambda i: (0, 0)),
            pl.BlockSpec((1, 1), lambda i: (0, 0)),
        ],
        out_specs=[
            pl.BlockSpec((T, 1), lambda i: (0, 0)),
            pl.BlockSpec((T, 16), lambda i: (0, 0)),
        ],
        out_shape=[
            jax.ShapeDtypeStruct((T, 1), jnp.int32),
            jax.ShapeDtypeStruct((T, 16), jnp.float32),
        ],
        scratch_shapes=[
            pltpu.VMEM((T, D), jnp.float32),
            pltpu.VMEM((T, 128), jnp.bfloat16),
            pltpu.VMEM((T, 1), jnp.float32),
            pltpu.VMEM((T, 1), jnp.int32),
        ],
    )(h_clean, features, pp_row, g_row, pp_hist, g_hist, phones2, tgt)


def _sc_gather(features, h_clean, idx, validf):
    info = plsc.get_sparse_core_info()
    nw = info.num_cores * info.num_subcores
    bpw = T // nw
    mesh = plsc.VectorSubcoreMesh(core_axis_name="c", subcore_axis_name="s")

    @functools.partial(
        pl.kernel,
        mesh=mesh,
        out_type=jax.ShapeDtypeStruct((T, D), jnp.float32),
        scratch_types=[
            pltpu.VMEM((bpw,), jnp.int32),
            pltpu.VMEM((bpw, D), jnp.float32),
            pltpu.VMEM((bpw, D), jnp.float32),
            pltpu.VMEM((bpw, 16), jnp.float32),
            pltpu.SemaphoreType.DMA,
        ],
    )
    def gather_k(feat_hbm, hcl_hbm, idx_hbm, val_hbm, out_hbm,
                 idx_v, rows_v, hcl_v, val_v, sem):
        wid = lax.axis_index("s") * info.num_cores + lax.axis_index("c")
        base = wid * bpw
        pltpu.sync_copy(idx_hbm.at[pl.ds(base, bpw)], idx_v)
        cp = pltpu.async_copy(feat_hbm.at[idx_v], rows_v, sem)
        pltpu.sync_copy(hcl_hbm.at[pl.ds(base, bpw)], hcl_v)
        pltpu.sync_copy(val_hbm.at[pl.ds(base, bpw)], val_v)
        cp.wait()

        def row(r, carry):
            vb = val_v[r, :] > 0.5
            for c in range(D // 16):
                g = rows_v[r, pl.ds(c * 16, 16)]
                h = hcl_v[r, pl.ds(c * 16, 16)]
                rows_v[r, pl.ds(c * 16, 16)] = jnp.where(vb, g, h)
            return carry

        lax.fori_loop(0, bpw, row, 0)
        pltpu.sync_copy(rows_v, out_hbm.at[pl.ds(base, bpw)])

    return gather_k(features, h_clean, idx, validf)


def kernel(h_clean, features, phones, pool_phones, genders, target_gender):
    h_clean = h_clean.astype(jnp.float32)
    features = features.astype(jnp.float32)
    pad = NPAD - N
    pp_pad = jnp.concatenate(
        [pool_phones.astype(jnp.int32), jnp.full((pad,), -1, jnp.int32)]
    )
    g_pad = jnp.concatenate(
        [genders.astype(jnp.int32), jnp.full((pad,), -1, jnp.int32)]
    )
    phones2 = phones.astype(jnp.int32).reshape(T, 1)
    tgt = jnp.asarray(target_gender, jnp.int32).reshape(1, 1)

    idx, validf = _tc_argmax(
        h_clean, features,
        pp_pad.reshape(1, NPAD), g_pad.reshape(1, NPAD),
        pp_pad.reshape(HR, 128), g_pad.reshape(HR, 128),
        phones2, tgt,
    )
    return _sc_gather(features, h_clean, idx.reshape(T), validf)


# speedup vs baseline: 1.0329x; 1.0329x over previous
import functools

import jax
import jax.numpy as jnp
from jax import lax
from jax.experimental import pallas as pl
from jax.experimental.pallas import tpu as pltpu
from jax.experimental.pallas import tpu_sc as plsc

T = 1024
D = 512
N = 50000
P = 64
K_NEAREST = 4
TN = 3584
NPAD = 50176
GRID = NPAD // TN
HR = NPAD // 128


def _tc_body(h_ref, f_ref, ppr_ref, gr_ref, pph_ref, gh_ref, ph_ref, tgt_ref,
             idx_out, val_out, hn_ref, q_ref, bv_ref, bi_ref):
    i = pl.program_id(0)
    tgt = tgt_ref[0, 0]

    @pl.when(i == 0)
    def _prologue():
        h = h_ref[...]
        hn_ref[...] = h / (jnp.sqrt(jnp.sum(h * h, axis=1, keepdims=True)) + 1e-8)
        bv_ref[...] = jnp.full((T, 1), -jnp.inf, jnp.float32)
        bi_ref[...] = jnp.zeros((T, 1), jnp.int32)
        gvm = gh_ref[...] == tgt
        ppm = jnp.where(gvm, pph_ref[...], -1)
        li1 = lax.broadcasted_iota(jnp.int32, (1, 128), 1)
        hrow = jnp.zeros((1, 128), jnp.float32)
        for p in range(P):
            cnt = jnp.sum((ppm == p).astype(jnp.float32))
            hrow = hrow + jnp.where(li1 == p, cnt, 0.0)
        ph = ph_ref[...]
        li = lax.broadcasted_iota(jnp.int32, (T, 128), 1)
        oh = (li == ph).astype(jnp.float32)
        counts = lax.dot_general(oh, hrow, (((1,), (1,)), ((), ())),
                                 preferred_element_type=jnp.float32)
        up = counts >= K_NEAREST
        q_ref[...] = jnp.where(up, oh, (li == P).astype(jnp.float32))

    f = f_ref[...]
    cn = f / (jnp.sqrt(jnp.sum(f * f, axis=1, keepdims=True)) + 1e-8)
    sim = lax.dot_general(hn_ref[...], cn, (((1,), (1,)), ((), ())),
                          preferred_element_type=jnp.float32)
    ppr = ppr_ref[0:1, pl.ds(i * TN, TN)]
    gvr = gr_ref[0:1, pl.ds(i * TN, TN)] == tgt
    ch = lax.broadcasted_iota(jnp.int32, (128, TN), 0)
    code = (((ppr == ch) | (ch == P)) & gvr).astype(jnp.float32)
    msk = lax.dot_general(q_ref[...], code, (((1,), (0,)), ((), ())),
                          preferred_element_type=jnp.float32)
    simm = jnp.where(msk > 0.5, sim, -jnp.inf)
    tmax = jnp.max(simm, axis=1, keepdims=True)
    li = lax.broadcasted_iota(jnp.int32, (T, TN), 1)
    tidx = jnp.min(jnp.where(simm == tmax, li, TN), axis=1, keepdims=True) + i * TN
    better = tmax > bv_ref[...]
    bv_ref[...] = jnp.where(better, tmax, bv_ref[...])
    bi_ref[...] = jnp.where(better, tidx, bi_ref[...])

    @pl.when(i == GRID - 1)
    def _epilogue():
        valid = bv_ref[...] > -jnp.inf
        idx_out[...] = jnp.where(valid, bi_ref[...], 0)
        val_out[...] = jnp.broadcast_to(valid.astype(jnp.float32), (T, 16))


def _tc_argmax(h_clean, features, pp_row, g_row, pp_hist, g_hist, phones2, tgt):
    return pl.pallas_call(
        _tc_body,
        grid=(GRID,),
        in_specs=[
            pl.BlockSpec((T, D), lambda i: (0, 0)),
            pl.BlockSpec((TN, D), lambda i: (i, 0)),
            pl.BlockSpec((1, NPAD), lambda i: (0, 0)),
            pl.BlockSpec((1, NPAD), lambda i: (0, 0)),
            pl.BlockSpec((HR, 128), lambda i: (0, 0)),
            pl.BlockSpec((HR, 128), lambda i: (0, 0)),
            pl.BlockSpec((T, 1), lambda i: (0, 0)),
            pl.BlockSpec((1, 1), lambda i: (0, 0)),
        ],
        out_specs=[
            pl.BlockSpec((T, 1), lambda i: (0, 0)),
            pl.BlockSpec((T, 16), lambda i: (0, 0)),
        ],
        out_shape=[
            jax.ShapeDtypeStruct((T, 1), jnp.int32),
            jax.ShapeDtypeStruct((T, 16), jnp.float32),
        ],
        scratch_shapes=[
            pltpu.VMEM((T, D), jnp.float32),
            pltpu.VMEM((T, 128), jnp.float32),
            pltpu.VMEM((T, 1), jnp.float32),
            pltpu.VMEM((T, 1), jnp.int32),
        ],
    )(h_clean, features, pp_row, g_row, pp_hist, g_hist, phones2, tgt)


def _sc_gather(features, h_clean, idx, validf):
    info = plsc.get_sparse_core_info()
    nw = info.num_cores * info.num_subcores
    bpw = T // nw
    mesh = plsc.VectorSubcoreMesh(core_axis_name="c", subcore_axis_name="s")

    @functools.partial(
        pl.kernel,
        mesh=mesh,
        out_type=jax.ShapeDtypeStruct((T, D), jnp.float32),
        scratch_types=[
            pltpu.VMEM((bpw,), jnp.int32),
            pltpu.VMEM((bpw, D), jnp.float32),
            pltpu.VMEM((bpw, D), jnp.float32),
            pltpu.VMEM((bpw, 16), jnp.float32),
            pltpu.SemaphoreType.DMA,
        ],
    )
    def gather_k(feat_hbm, hcl_hbm, idx_hbm, val_hbm, out_hbm,
                 idx_v, rows_v, hcl_v, val_v, sem):
        wid = lax.axis_index("s") * info.num_cores + lax.axis_index("c")
        base = wid * bpw
        pltpu.sync_copy(idx_hbm.at[pl.ds(base, bpw)], idx_v)
        cp = pltpu.async_copy(feat_hbm.at[idx_v], rows_v, sem)
        pltpu.sync_copy(hcl_hbm.at[pl.ds(base, bpw)], hcl_v)
        pltpu.sync_copy(val_hbm.at[pl.ds(base, bpw)], val_v)
        cp.wait()

        def row(r, carry):
            vb = val_v[r, :] > 0.5
            for c in range(D // 16):
                g = rows_v[r, pl.ds(c * 16, 16)]
                h = hcl_v[r, pl.ds(c * 16, 16)]
                rows_v[r, pl.ds(c * 16, 16)] = jnp.where(vb, g, h)
            return carry

        lax.fori_loop(0, bpw, row, 0)
        pltpu.sync_copy(rows_v, out_hbm.at[pl.ds(base, bpw)])

    return gather_k(features, h_clean, idx, validf)


def kernel(h_clean, features, phones, pool_phones, genders, target_gender):
    h_clean = h_clean.astype(jnp.float32)
    features = features.astype(jnp.float32)
    pad = NPAD - N
    pp_pad = jnp.concatenate(
        [pool_phones.astype(jnp.int32), jnp.full((pad,), -1, jnp.int32)]
    )
    g_pad = jnp.concatenate(
        [genders.astype(jnp.int32), jnp.full((pad,), -1, jnp.int32)]
    )
    phones2 = phones.astype(jnp.int32).reshape(T, 1)
    tgt = jnp.asarray(target_gender, jnp.int32).reshape(1, 1)

    idx, validf = _tc_argmax(
        h_clean, features,
        pp_pad.reshape(1, NPAD), g_pad.reshape(1, NPAD),
        pp_pad.reshape(HR, 128), g_pad.reshape(HR, 128),
        phones2, tgt,
    )
    return _sc_gather(features, h_clean, idx.reshape(T), validf)
